# baseline (speedup 1.0000x reference)
import functools
import math

import jax
import jax.numpy as jnp
from jax import lax
from jax.experimental import pallas as pl
from jax.experimental.pallas import tpu as pltpu
from jax.experimental.pallas import tpu_sc as plsc

N = 4096
D = 64
L = 200
R = 8
M = 2 * N
NW = 32
RPW = 7
MP = M + 16
NBINS = 2048
NLEV = 2 * N + 1
NLEVP = 513 * 16


def _keys_block(u0_ref, u1_ref, xt_ref, yt_ref, out_ref):
    xt = xt_ref[...]
    yt = yt_ref[...]
    u0 = u0_ref[...]
    u1 = u1_ref[...]

    xa = jnp.dot(u0, xt, preferred_element_type=jnp.float32)
    xb = jnp.dot(u1, xt, preferred_element_type=jnp.float32)
    ya = jnp.dot(u0, yt, preferred_element_type=jnp.float32)
    yb = jnp.dot(u1, yt, preferred_element_type=jnp.float32)

    two_pi_inv = 1.0 / (2.0 * math.pi)
    ax = (jnp.arctan2(-xb, -xa) + math.pi) * two_pi_inv
    ay = (jnp.arctan2(-yb, -ya) + math.pi) * two_pi_inv

    kx = pltpu.bitcast(ax, jnp.int32) | 1
    ky = pltpu.bitcast(ay, jnp.int32) & ~1
    out_ref[...] = jnp.concatenate([kx, ky], axis=1)


def _digits(v, shift):
    return lax.shift_right_logical(v, shift) & (NBINS - 1)


def _sc_body(keys_hbm, out_hbm, ka, kb, hist, hlev, wv):
    wid = lax.axis_index("s") * 2 + lax.axis_index("c")
    wv[...] = jnp.zeros((16,), jnp.float32)
    iota16 = lax.broadcasted_iota(jnp.int32, (16,), 0)

    def zero_hist(c, _):
        hist[pl.ds(c * 16, 16)] = jnp.zeros((16,), jnp.int32)
        return 0

    def radix_pass(kin, kout, shift):
        lax.fori_loop(0, NBINS // 16, zero_hist, 0)

        def hist_body(c, _):
            d = _digits(kin[pl.ds(c * 16, 16)], shift)
            plsc.addupdate_scatter(hist, [d], jnp.ones((16,), jnp.int32))
            return 0
        lax.fori_loop(0, M // 16, hist_body, 0)

        def scan_body(c, carry):
            h = hist[pl.ds(c * 16, 16)]
            hist[pl.ds(c * 16, 16)] = plsc.cumsum(h) - h + carry
            return carry + jnp.sum(h)
        lax.fori_loop(0, NBINS // 16, scan_body, jnp.int32(0))

        def perm_body(c, _):
            v = kin[pl.ds(c * 16, 16)]
            d = _digits(v, shift)
            cnt, lastm = plsc.scan_count(d)
            base = plsc.load_gather(hist, [d])
            plsc.store_scatter(kout, [base + cnt - 1], v)
            plsc.addupdate_scatter(hist, [d], cnt, mask=lastm)
            return 0
        lax.fori_loop(0, M // 16, perm_body, 0)

    def row_body(r, _):
        row = r * NW + wid

        @pl.when(row < L)
        def _():
            pltpu.sync_copy(keys_hbm.at[row], ka.at[pl.ds(0, M)])
            radix_pass(ka, kb, 0)
            radix_pass(kb, ka, 11)
            radix_pass(ka, kb, 22)
            kb[pl.ds(M, 16)] = jnp.full((16,), 0x3F800000, jnp.int32)

            def zero_lev(c, _):
                hlev[pl.ds(c * 16, 16)] = jnp.zeros((16,), jnp.float32)
                return 0
            lax.fori_loop(0, NLEVP // 16, zero_lev, 0)

            def main_body(c, carry):
                cdfc, tot16, mnc = carry
                v = kb[pl.ds(c * 16, 16)]
                val = plsc.bitcast(v, jnp.float32)
                nxt = plsc.bitcast(kb[pl.ds(c * 16 + 1, 16)], jnp.float32)
                delta = nxt - val
                sgn = 2 * (v & 1) - 1
                cdf = plsc.cumsum(sgn) + cdfc
                plsc.addupdate_scatter(hlev, [cdf + N], delta)
                return (cdfc + jnp.sum(sgn), tot16 + delta,
                        jnp.minimum(mnc, jnp.min(cdf)))
            _, tot16, mnc = lax.fori_loop(
                0, M // 16, main_body,
                (jnp.int32(0), jnp.zeros((16,), jnp.float32), jnp.int32(N)))
            tot = jnp.sum(tot16)

            def med_body(c, carry):
                cum16, nbefore = carry
                h = hlev[pl.ds(c * 16, 16)]
                cs = plsc.cumsum(h) + cum16
                nbefore = nbefore + jnp.sum(
                    jnp.where(cs < 0.5, 1, 0).astype(jnp.int32))
                return (cum16 + jnp.full((16,), jnp.sum(h), jnp.float32),
                        nbefore)
            _, nbefore = lax.fori_loop(
                0, NLEVP // 16, med_body,
                (jnp.zeros((16,), jnp.float32), jnp.int32(0)))

            ok16 = jnp.full((16,), tot, jnp.float32) >= 0.5
            med16 = jnp.where(ok16, jnp.full((16,), nbefore - N, jnp.int32),
                              jnp.full((16,), mnc, jnp.int32))

            def fin_body(c, w16):
                h = hlev[pl.ds(c * 16, 16)]
                lvl = iota16 + (c * 16 - N)
                return w16 + h * jnp.abs(lvl - med16).astype(jnp.float32)
            w16 = lax.fori_loop(0, NLEVP // 16, fin_body,
                                jnp.zeros((16,), jnp.float32))
            wval = jnp.sum(w16 * jnp.float32(1.0 / N))
            wv[...] = jnp.where(iota16 == r, jnp.full((16,), wval, jnp.float32),
                                wv[...])
        return 0

    lax.fori_loop(0, RPW, row_body, 0)
    pltpu.sync_copy(wv, out_hbm.at[wid])


@jax.jit
def kernel(x, y, U):
    xt = x.T
    yt = y.T
    u0 = U[:, :, 0]
    u1 = U[:, :, 1]

    nb = L // R
    keys = pl.pallas_call(
        _keys_block,
        grid=(nb,),
        in_specs=[
            pl.BlockSpec((R, D), lambda i: (i, 0)),
            pl.BlockSpec((R, D), lambda i: (i, 0)),
            pl.BlockSpec((D, N), lambda i: (0, 0)),
            pl.BlockSpec((D, N), lambda i: (0, 0)),
        ],
        out_specs=pl.BlockSpec((R, M), lambda i: (i, 0)),
        out_shape=jax.ShapeDtypeStruct((L, M), jnp.int32),
    )(u0, u1, xt, yt)

    mesh = plsc.VectorSubcoreMesh(core_axis_name="c", subcore_axis_name="s")
    sc = functools.partial(
        pl.kernel,
        out_type=jax.ShapeDtypeStruct((NW, 16), jnp.float32),
        mesh=mesh,
        compiler_params=pltpu.CompilerParams(needs_layout_passes=False),
        scratch_types=[
            pltpu.VMEM((MP,), jnp.int32),
            pltpu.VMEM((MP,), jnp.int32),
            pltpu.VMEM((NBINS,), jnp.int32),
            pltpu.VMEM((NLEVP,), jnp.float32),
            pltpu.VMEM((16,), jnp.float32),
        ],
    )(_sc_body)
    wout = sc(keys)

    w = wout.T[:RPW].reshape(-1)[:L]
    return jnp.max(w)


# speedup vs baseline: 1.9848x; 1.9848x over previous
import functools
import math

import jax
import jax.numpy as jnp
from jax import lax
from jax.experimental import pallas as pl
from jax.experimental.pallas import tpu as pltpu
from jax.experimental.pallas import tpu_sc as plsc

N = 4096
D = 64
L = 200
M = 2 * N
R = 8

LS = 96
NW = 32
RPW = LS // NW
LT = L - LS

MP = M + 16
NBINS = 2048
NLEVP = 513 * 16


def _angles(u0, u1, xt, yt):
    xa = jnp.dot(u0, xt, preferred_element_type=jnp.float32)
    xb = jnp.dot(u1, xt, preferred_element_type=jnp.float32)
    ya = jnp.dot(u0, yt, preferred_element_type=jnp.float32)
    yb = jnp.dot(u1, yt, preferred_element_type=jnp.float32)
    two_pi_inv = 1.0 / (2.0 * math.pi)
    ax = (jnp.arctan2(-xb, -xa) + math.pi) * two_pi_inv
    ay = (jnp.arctan2(-yb, -ya) + math.pi) * two_pi_inv
    return ax, ay


def _tagged_keys(ax, ay):
    kx = pltpu.bitcast(ax, jnp.int32) | 1
    ky = pltpu.bitcast(ay, jnp.int32) & ~1
    return jnp.concatenate([kx, ky], axis=1)


def _keys_block(u0_ref, u1_ref, xt_ref, yt_ref, out_ref):
    ax, ay = _angles(u0_ref[...], u1_ref[...], xt_ref[...], yt_ref[...])
    out_ref[...] = _tagged_keys(ax, ay)


def _digits(v, shift):
    return lax.shift_right_logical(v, shift) & (NBINS - 1)


def _sc_body(keys_hbm, out_hbm, ka, kb, hist, hlev, wv):
    wid = lax.axis_index("s") * 2 + lax.axis_index("c")
    wv[...] = jnp.zeros((16,), jnp.float32)
    iota16 = lax.broadcasted_iota(jnp.int32, (16,), 0)

    def zero_hist(c, _):
        hist[pl.ds(c * 16, 16)] = jnp.zeros((16,), jnp.int32)
        return 0

    def zero_lev(c, _):
        hlev[pl.ds(c * 16, 16)] = jnp.zeros((16,), jnp.float32)
        return 0

    lax.fori_loop(0, NLEVP // 16, zero_lev, 0)

    def radix_pass(kin, kout, shift):
        lax.fori_loop(0, NBINS // 16, zero_hist, 0)

        def hist_body(c, _):
            d = _digits(kin[pl.ds(c * 16, 16)], shift)
            plsc.addupdate_scatter(hist, [d], jnp.ones((16,), jnp.int32))
            return 0
        lax.fori_loop(0, M // 16, hist_body, 0)

        def scan_body(c, carry):
            h = hist[pl.ds(c * 16, 16)]
            hist[pl.ds(c * 16, 16)] = plsc.cumsum(h) - h + carry
            return carry + jnp.sum(h)
        lax.fori_loop(0, NBINS // 16, scan_body, jnp.int32(0))

        def perm_body(c, _):
            v = kin[pl.ds(c * 16, 16)]
            d = _digits(v, shift)
            cnt, lastm = plsc.scan_count(d)
            base = plsc.load_gather(hist, [d])
            plsc.store_scatter(kout, [base + cnt - 1], v)
            plsc.addupdate_scatter(hist, [d], cnt, mask=lastm)
            return 0
        lax.fori_loop(0, M // 16, perm_body, 0)

    def row_body(r, _):
        row = r * NW + wid

        @pl.when(row < LS)
        def _():
            pltpu.sync_copy(keys_hbm.at[row], ka.at[pl.ds(0, M)])
            radix_pass(ka, kb, 0)
            radix_pass(kb, ka, 11)
            radix_pass(ka, kb, 22)
            kb[pl.ds(M, 16)] = jnp.full((16,), 0x3F800000, jnp.int32)

            def main_body(c, carry):
                cdfc, tot16, mnc, mxc = carry
                v = kb[pl.ds(c * 16, 16)]
                val = plsc.bitcast(v, jnp.float32)
                nxt = plsc.bitcast(kb[pl.ds(c * 16 + 1, 16)], jnp.float32)
                delta = nxt - val
                sgn = 2 * (v & 1) - 1
                cdf = plsc.cumsum(sgn) + cdfc
                plsc.addupdate_scatter(hlev, [cdf + N], delta)
                return (cdfc + jnp.sum(sgn), tot16 + delta,
                        jnp.minimum(mnc, jnp.min(cdf)),
                        jnp.maximum(mxc, jnp.max(cdf)))
            _, tot16, mnc, mxc = lax.fori_loop(
                0, M // 16, main_body,
                (jnp.int32(0), jnp.zeros((16,), jnp.float32),
                 jnp.int32(N), jnp.int32(-N)))
            tot = jnp.sum(tot16)

            c0 = (mnc + N) // 16
            c1 = (mxc + N) // 16 + 1

            def med_body(c, carry):
                cum16, nbefore = carry
                h = hlev[pl.ds(c * 16, 16)]
                cs = plsc.cumsum(h) + cum16
                nbefore = nbefore + jnp.sum(
                    jnp.where(cs < 0.5, 1, 0).astype(jnp.int32))
                return (cum16 + jnp.full((16,), jnp.sum(h), jnp.float32),
                        nbefore)
            _, nbefore = lax.fori_loop(
                c0, c1, med_body, (jnp.zeros((16,), jnp.float32),
                                   jnp.int32(0)))

            ok16 = jnp.full((16,), tot, jnp.float32) >= 0.5
            med16 = jnp.where(
                ok16, jnp.full((16,), c0 * 16 + nbefore - N, jnp.int32),
                jnp.full((16,), mnc, jnp.int32))

            def fin_body(c, w16):
                h = hlev[pl.ds(c * 16, 16)]
                lvl = iota16 + (c * 16 - N)
                w16 = w16 + h * jnp.abs(lvl - med16).astype(jnp.float32)
                hlev[pl.ds(c * 16, 16)] = jnp.zeros((16,), jnp.float32)
                return w16
            w16 = lax.fori_loop(c0, c1, fin_body,
                                jnp.zeros((16,), jnp.float32))
            wval = jnp.sum(w16 * jnp.float32(1.0 / N))
            wv[...] = jnp.where(iota16 == r,
                                jnp.full((16,), wval, jnp.float32), wv[...])
        return 0

    lax.fori_loop(0, RPW, row_body, 0)
    pltpu.sync_copy(wv, out_hbm.at[wid])


def _w1_block(u0_ref, u1_ref, xt_ref, yt_ref, out_ref):
    ax, ay = _angles(u0_ref[...], u1_ref[...], xt_ref[...], yt_ref[...])
    keys = _tagged_keys(ax, ay)

    iota = lax.broadcasted_iota(jnp.int32, (R, M), 1)

    prev_flip = jnp.zeros((R, M), jnp.int32)
    k = 2
    while k <= M:
        flip = jnp.where((iota & k) == 0, 0, -1)
        keys = keys ^ (prev_flip ^ flip)
        prev_flip = flip
        j = k >> 1
        while j >= 1:
            down = pltpu.roll(keys, M - j, 1)
            up = pltpu.roll(keys, j, 1)
            bitj0 = (iota & j) == 0
            mn = jnp.minimum(keys, down)
            mx = jnp.maximum(keys, up)
            keys = jnp.where(bitj0, mn, mx)
            j >>= 1
        k <<= 1
    keys = keys ^ prev_flip

    val = pltpu.bitcast(keys, jnp.float32)
    sgn = 2 * (keys & 1) - 1

    cdf = sgn
    sh = 1
    while sh < M:
        cdf = cdf + jnp.where(iota >= sh, pltpu.roll(cdf, sh, 1), 0)
        sh <<= 1

    nxt = jnp.where(iota == M - 1, 1.0, pltpu.roll(val, M - 1, 1))
    delta = nxt - val

    total = jnp.sum(delta, axis=1, keepdims=True)
    mincdf = jnp.min(cdf, axis=1, keepdims=True)

    lo = jnp.full((R, 1), -N, jnp.int32)
    hi = jnp.full((R, 1), N, jnp.int32)
    for _ in range(13):
        mid = lax.shift_right_arithmetic(lo + hi, 1)
        fmid = jnp.sum(jnp.where(cdf <= mid, delta, 0.0), axis=1, keepdims=True)
        ok = fmid >= 0.5
        hi = jnp.where(ok, mid, hi)
        lo = jnp.where(ok, lo, mid + 1)
    med = jnp.where(total >= 0.5, lo, mincdf)

    dev = jnp.abs(cdf - med).astype(jnp.float32)
    w = jnp.sum(delta * dev, axis=1) * (1.0 / N)
    out_ref[0, 0, :] = w


@jax.jit
def kernel(x, y, U):
    xt = x.T
    yt = y.T
    u0 = U[:, :, 0]
    u1 = U[:, :, 1]

    nbs = LS // R
    keys = pl.pallas_call(
        _keys_block,
        grid=(nbs,),
        in_specs=[
            pl.BlockSpec((R, D), lambda i: (i, 0)),
            pl.BlockSpec((R, D), lambda i: (i, 0)),
            pl.BlockSpec((D, N), lambda i: (0, 0)),
            pl.BlockSpec((D, N), lambda i: (0, 0)),
        ],
        out_specs=pl.BlockSpec((R, M), lambda i: (i, 0)),
        out_shape=jax.ShapeDtypeStruct((LS, M), jnp.int32),
    )(u0, u1, xt, yt)

    mesh = plsc.VectorSubcoreMesh(core_axis_name="c", subcore_axis_name="s")
    wsc = functools.partial(
        pl.kernel,
        out_type=jax.ShapeDtypeStruct((NW, 16), jnp.float32),
        mesh=mesh,
        compiler_params=pltpu.CompilerParams(needs_layout_passes=False),
        scratch_types=[
            pltpu.VMEM((MP,), jnp.int32),
            pltpu.VMEM((MP,), jnp.int32),
            pltpu.VMEM((NBINS,), jnp.int32),
            pltpu.VMEM((NLEVP,), jnp.float32),
            pltpu.VMEM((16,), jnp.float32),
        ],
    )(_sc_body)(keys)

    nbt = LT // R
    wtc = pl.pallas_call(
        _w1_block,
        grid=(nbt,),
        in_specs=[
            pl.BlockSpec((R, D), lambda i: (i + LS // R, 0)),
            pl.BlockSpec((R, D), lambda i: (i + LS // R, 0)),
            pl.BlockSpec((D, N), lambda i: (0, 0)),
            pl.BlockSpec((D, N), lambda i: (0, 0)),
        ],
        out_specs=pl.BlockSpec((1, 1, R), lambda i: (i, 0, 0)),
        out_shape=jax.ShapeDtypeStruct((nbt, 1, R), jnp.float32),
    )(u0, u1, xt, yt)

    wsc_flat = wsc.T[:RPW].reshape(-1)
    return jnp.maximum(jnp.max(wsc_flat), jnp.max(wtc))
